# baseline (speedup 1.0000x reference)
import functools

import jax
import jax.numpy as jnp
from jax import lax
from jax.experimental import pallas as pl
from jax.experimental.pallas import tpu as pltpu
from jax.experimental.pallas import tpu_sc as plsc

_B, _P, _CIN, _COUT = 8, 2048, 128, 256
_M, _K = 512, 16
_N = _B * _P
_EPS = 1e-5
_NW = 32
_NB = (_B * _M) // _NW


_NC = _P // 128


def _fps_body(px_ref, py_ref, pz_ref, cx_ref, cy_ref, cz_ref):
    ixs = [px_ref[:, 128 * c:128 * (c + 1)] for c in range(_NC)]
    iys = [py_ref[:, 128 * c:128 * (c + 1)] for c in range(_NC)]
    izs = [pz_ref[:, 128 * c:128 * (c + 1)] for c in range(_NC)]
    lane = lax.broadcasted_iota(jnp.int32, (_B, 128), 1)
    mlane = lax.broadcasted_iota(jnp.int32, (_B, _M), 1)
    x0 = ixs[0][:, 0:1]
    y0 = iys[0][:, 0:1]
    z0 = izs[0][:, 0:1]
    dists = []
    for c in range(_NC):
        d0x = ixs[c] - x0
        d0y = iys[c] - y0
        d0z = izs[c] - z0
        dists.append(d0x * d0x + d0y * d0y + d0z * d0z)
    cx = jnp.where(mlane == 0, jnp.broadcast_to(x0, (_B, _M)), 0.0)
    cy = jnp.where(mlane == 0, jnp.broadcast_to(y0, (_B, _M)), 0.0)
    cz = jnp.where(mlane == 0, jnp.broadcast_to(z0, (_B, _M)), 0.0)

    def _tmax(vals):
        while len(vals) > 1:
            vals = [jnp.maximum(vals[2 * i], vals[2 * i + 1])
                    for i in range(len(vals) // 2)]
        return vals[0]

    def _pick(mask_list, vals, fill):
        sel = [jnp.where(mask_list[c], vals[c], fill) for c in range(_NC)]
        return _tmax(sel)

    def body(i, carry):
        dists, cx, cy, cz = carry
        mc = [jnp.max(dists[c], axis=1, keepdims=True) for c in range(_NC)]
        m = _tmax(list(mc))
        cidx = [jnp.where(mc[c] == m, c, _NC) for c in range(_NC)]
        cstar = cidx[0]
        for c in range(1, _NC):
            cstar = jnp.minimum(cstar, cidx[c])
        cmask = [cstar == c for c in range(_NC)]
        vstar = _pick(cmask, dists, -1.0)
        lstar = jnp.min(jnp.where(vstar == m, lane, 128), axis=1,
                        keepdims=True)
        selm = lane == lstar
        pxs = _pick(cmask, ixs, -1.0)
        pys = _pick(cmask, iys, -1.0)
        pzs = _pick(cmask, izs, -1.0)
        lx = jnp.max(jnp.where(selm, pxs, -1.0), axis=1, keepdims=True)
        ly = jnp.max(jnp.where(selm, pys, -1.0), axis=1, keepdims=True)
        lz = jnp.max(jnp.where(selm, pzs, -1.0), axis=1, keepdims=True)
        new = []
        for c in range(_NC):
            dx = ixs[c] - lx
            dy = iys[c] - ly
            dz = izs[c] - lz
            dn = dx * dx + dy * dy + dz * dz
            new.append(jnp.minimum(dists[c], dn))
        upd = mlane == i
        cx = jnp.where(upd, lx, cx)
        cy = jnp.where(upd, ly, cy)
        cz = jnp.where(upd, lz, cz)
        return (new, cx, cy, cz)

    _, cx, cy, cz = lax.fori_loop(1, _M, body, (dists, cx, cy, cz))
    cx_ref[...] = cx
    cy_ref[...] = cy
    cz_ref[...] = cz


def _fps(px, py, pz):
    out = jax.ShapeDtypeStruct((_B, _M), jnp.float32)
    return pl.pallas_call(
        _fps_body,
        out_shape=(out, out, out),
    )(px, py, pz)


_RB = 1024
_NG = _N // _RB


def _mlp_body(x_ref, w_ref, b_ref, g_ref, be_ref, h_ref, ss_ref):
    i = pl.program_id(0)
    h = jnp.dot(x_ref[...], w_ref[...],
                preferred_element_type=jnp.float32,
                precision=lax.Precision.HIGHEST) + b_ref[...]
    h_ref[...] = h
    s1 = jnp.sum(h, axis=0, keepdims=True)
    s2 = jnp.sum(h * h, axis=0, keepdims=True)

    @pl.when(i == 0)
    def _():
        ss_ref[2:3, :] = s1
        ss_ref[3:4, :] = s2

    @pl.when(i > 0)
    def _():
        ss_ref[2:3, :] = ss_ref[2:3, :] + s1
        ss_ref[3:4, :] = ss_ref[3:4, :] + s2

    @pl.when(i == _NG - 1)
    def _():
        mu = ss_ref[2:3, :] / _N
        var = ss_ref[3:4, :] / _N - mu * mu
        scale = g_ref[...] / jnp.sqrt(var + _EPS)
        ss_ref[0:1, :] = scale
        ss_ref[1:2, :] = be_ref[...] - mu * scale


def _mlp(x, W, b, gamma, beta):
    return pl.pallas_call(
        _mlp_body,
        grid=(_NG,),
        in_specs=[
            pl.BlockSpec((_RB, _CIN), lambda i: (i, 0)),
            pl.BlockSpec((_CIN, _COUT), lambda i: (0, 0)),
            pl.BlockSpec((1, _COUT), lambda i: (0, 0)),
            pl.BlockSpec((1, _COUT), lambda i: (0, 0)),
            pl.BlockSpec((1, _COUT), lambda i: (0, 0)),
        ],
        out_specs=(
            pl.BlockSpec((_RB, _COUT), lambda i: (i, 0)),
            pl.BlockSpec((8, _COUT), lambda i: (0, 0)),
        ),
        out_shape=(
            jax.ShapeDtypeStruct((_N, _COUT), jnp.float32),
            jax.ShapeDtypeStruct((8, _COUT), jnp.float32),
        ),
    )(x, W, b.reshape(1, _COUT), gamma.reshape(1, _COUT),
      beta.reshape(1, _COUT))


def _knn_body(px_ref, py_ref, pz_ref, cx_ref, cy_ref, cz_ref, col_ref):
    b = pl.program_id(0)
    px = px_ref[0]
    py = py_ref[0]
    pz = pz_ref[0]
    gw = 512
    lane = lax.broadcasted_iota(jnp.int32, (gw, _P), 1)
    base = b * _P

    def group(g, _):
        cxg = cx_ref[0, pl.ds(g * gw, gw), :]
        cyg = cy_ref[0, pl.ds(g * gw, gw), :]
        czg = cz_ref[0, pl.ds(g * gw, gw), :]
        dx = cxg - px
        dy = cyg - py
        dz = czg - pz
        d = dx * dx + dy * dy + dz * dz
        cols = []
        for _k in range(_K):
            mv = jnp.min(d, axis=1, keepdims=True)
            j = jnp.min(jnp.where(d == mv, lane, _P), axis=1, keepdims=True)
            cols.append(j)
            d = jnp.where(lane == j, jnp.inf, d)
        out = jnp.concatenate(cols, axis=1) + base
        col_ref[0, pl.ds(g * gw, gw), :] = out
        return 0

    lax.fori_loop(0, _M // gw, group, 0)


def _knn(px, py, pz, cx3, cy3, cz3):
    c_spec = pl.BlockSpec((1, _M, 1), lambda b: (b, 0, 0))
    p_spec = pl.BlockSpec((1, 1, _P), lambda b: (b, 0, 0))
    return pl.pallas_call(
        _knn_body,
        grid=(_B,),
        in_specs=[p_spec, p_spec, p_spec, c_spec, c_spec, c_spec],
        out_specs=pl.BlockSpec((1, _M, _K), lambda b: (b, 0, 0)),
        out_shape=jax.ShapeDtypeStruct((_B, _M, _K), jnp.int32),
    )(px.reshape(_B, 1, _P), py.reshape(_B, 1, _P), pz.reshape(_B, 1, _P),
      cx3, cy3, cz3)


def _pool(h, col, scale, shift):
    mesh = plsc.VectorSubcoreMesh(core_axis_name="c", subcore_axis_name="s")

    @functools.partial(
        pl.kernel,
        out_type=[
            jax.ShapeDtypeStruct((_B * _M, _COUT), jnp.float32),
            jax.ShapeDtypeStruct((_B * _M,), jnp.int32),
        ],
        mesh=mesh,
        scratch_types=[
            pltpu.VMEM((_NB, _K), jnp.int32),
            pltpu.VMEM((_K, _COUT), jnp.float32),
            pltpu.VMEM((_NB, _COUT), jnp.float32),
            pltpu.VMEM((_COUT,), jnp.float32),
            pltpu.VMEM((_COUT,), jnp.float32),
            pltpu.VMEM((_NB,), jnp.int32),
            pltpu.SemaphoreType.DMA,
        ],
    )
    def k(h_hbm, col_hbm, sc_hbm, sh_hbm, out_hbm, sb_hbm,
          idx_v, rows_v, out_v, sc_v, sh_v, sb_v, sem):
        wid = lax.axis_index("s") * 2 + lax.axis_index("c")
        base = wid * _NB
        pltpu.sync_copy(col_hbm.at[pl.ds(base, _NB)], idx_v)
        pltpu.sync_copy(sc_hbm, sc_v)
        pltpu.sync_copy(sh_hbm, sh_v)

        @pl.loop(0, _NB)
        def _(i):
            pltpu.async_copy(h_hbm.at[idx_v.at[i]], rows_v, sem).wait()
            for c in range(_COUT // 16):
                sl = pl.ds(c * 16, 16)
                acc = rows_v[0, sl]
                for r in range(1, _K):
                    acc = jnp.maximum(acc, rows_v[r, sl])
                y = jnp.maximum(acc * sc_v[sl] + sh_v[sl], 0.0)
                out_v[i, sl] = y

        @pl.loop(0, _NB, step=16)
        def _(i):
            v = lax.broadcasted_iota(jnp.int32, (16,), 0)
            sb_v[pl.ds(i, 16)] = lax.shift_right_logical(v + (base + i), 9)

        pltpu.sync_copy(out_v, out_hbm.at[pl.ds(base, _NB)])
        pltpu.sync_copy(sb_v, sb_hbm.at[pl.ds(base, _NB)])

    return k(h, col, scale, shift)


def kernel(x, pos, batch, W, b, gamma, beta):
    pos3 = pos.reshape(_B, _P, 3)
    px = pos3[:, :, 0]
    py = pos3[:, :, 1]
    pz = pos3[:, :, 2]
    cx, cy, cz = _fps(px, py, pz)
    h, ss = _mlp(x, W, b, gamma, beta)
    col3 = _knn(px, py, pz,
                cx.reshape(_B, _M, 1), cy.reshape(_B, _M, 1),
                cz.reshape(_B, _M, 1))
    col = col3.reshape(_B * _M, _K)
    x_out, sub_batch = _pool(h, col, ss[0], ss[1])
    sub_pos = jnp.stack([cx, cy, cz], axis=-1).reshape(_B * _M, 3)
    return (x_out, sub_pos, sub_batch)


# speedup vs baseline: 1.1732x; 1.1732x over previous
import functools

import jax
import jax.numpy as jnp
from jax import lax
from jax.experimental import pallas as pl
from jax.experimental.pallas import tpu as pltpu
from jax.experimental.pallas import tpu_sc as plsc

_B, _P, _CIN, _COUT = 8, 2048, 128, 256
_M, _K = 512, 16
_N = _B * _P
_EPS = 1e-5
_NW = 32
_NB = (_B * _M) // _NW


_NC = _P // 128


def _fps_body(px_ref, py_ref, pz_ref, cx_ref, cy_ref, cz_ref):
    ixs = [px_ref[:, 128 * c:128 * (c + 1)] for c in range(_NC)]
    iys = [py_ref[:, 128 * c:128 * (c + 1)] for c in range(_NC)]
    izs = [pz_ref[:, 128 * c:128 * (c + 1)] for c in range(_NC)]
    lane = lax.broadcasted_iota(jnp.int32, (_B, 128), 1)
    mlane = lax.broadcasted_iota(jnp.int32, (_B, _M), 1)
    x0 = ixs[0][:, 0:1]
    y0 = iys[0][:, 0:1]
    z0 = izs[0][:, 0:1]
    dists = []
    for c in range(_NC):
        d0x = ixs[c] - x0
        d0y = iys[c] - y0
        d0z = izs[c] - z0
        dists.append(d0x * d0x + d0y * d0y + d0z * d0z)
    cx = jnp.where(mlane == 0, jnp.broadcast_to(x0, (_B, _M)), 0.0)
    cy = jnp.where(mlane == 0, jnp.broadcast_to(y0, (_B, _M)), 0.0)
    cz = jnp.where(mlane == 0, jnp.broadcast_to(z0, (_B, _M)), 0.0)

    def _tmax(vals):
        while len(vals) > 1:
            vals = [jnp.maximum(vals[2 * i], vals[2 * i + 1])
                    for i in range(len(vals) // 2)]
        return vals[0]

    def _pick(mask_list, vals, fill):
        sel = [jnp.where(mask_list[c], vals[c], fill) for c in range(_NC)]
        return _tmax(sel)

    def body(i, carry):
        dists, cx, cy, cz = carry
        mc = [jnp.max(dists[c], axis=1, keepdims=True) for c in range(_NC)]
        m = _tmax(list(mc))
        cidx = [jnp.where(mc[c] == m, c, _NC) for c in range(_NC)]
        cstar = cidx[0]
        for c in range(1, _NC):
            cstar = jnp.minimum(cstar, cidx[c])
        cmask = [cstar == c for c in range(_NC)]
        vstar = _pick(cmask, dists, -1.0)
        lstar = jnp.min(jnp.where(vstar == m, lane, 128), axis=1,
                        keepdims=True)
        selm = lane == lstar
        pxs = _pick(cmask, ixs, -1.0)
        pys = _pick(cmask, iys, -1.0)
        pzs = _pick(cmask, izs, -1.0)
        lx = jnp.max(jnp.where(selm, pxs, -1.0), axis=1, keepdims=True)
        ly = jnp.max(jnp.where(selm, pys, -1.0), axis=1, keepdims=True)
        lz = jnp.max(jnp.where(selm, pzs, -1.0), axis=1, keepdims=True)
        new = []
        for c in range(_NC):
            dx = ixs[c] - lx
            dy = iys[c] - ly
            dz = izs[c] - lz
            dn = dx * dx + dy * dy + dz * dz
            new.append(jnp.minimum(dists[c], dn))
        upd = mlane == i
        cx = jnp.where(upd, lx, cx)
        cy = jnp.where(upd, ly, cy)
        cz = jnp.where(upd, lz, cz)
        return (new, cx, cy, cz)

    _, cx, cy, cz = lax.fori_loop(1, _M, body, (dists, cx, cy, cz))
    cx_ref[...] = cx
    cy_ref[...] = cy
    cz_ref[...] = cz


def _fps(px, py, pz):
    out = jax.ShapeDtypeStruct((_B, _M), jnp.float32)
    return pl.pallas_call(
        _fps_body,
        out_shape=(out, out, out),
    )(px, py, pz)


def _knn_body(px_ref, py_ref, pz_ref, cx_ref, cy_ref, cz_ref,
              x_ref, w_ref, b_ref, g_ref, be_ref, col_ref, h_ref, ss_ref):
    b = pl.program_id(0)
    h = jnp.dot(x_ref[...], w_ref[...],
                preferred_element_type=jnp.float32,
                precision=lax.Precision.HIGHEST) + b_ref[...]
    h_ref[...] = h
    s1 = jnp.sum(h, axis=0, keepdims=True)
    s2 = jnp.sum(h * h, axis=0, keepdims=True)

    @pl.when(b == 0)
    def _():
        ss_ref[2:3, :] = s1
        ss_ref[3:4, :] = s2

    @pl.when(b > 0)
    def _():
        ss_ref[2:3, :] = ss_ref[2:3, :] + s1
        ss_ref[3:4, :] = ss_ref[3:4, :] + s2

    @pl.when(b == _B - 1)
    def _():
        mu = ss_ref[2:3, :] / _N
        var = ss_ref[3:4, :] / _N - mu * mu
        scale = g_ref[...] / jnp.sqrt(var + _EPS)
        ss_ref[0:1, :] = scale
        ss_ref[1:2, :] = be_ref[...] - mu * scale

    px = px_ref[0]
    py = py_ref[0]
    pz = pz_ref[0]
    gw = 512
    lane = lax.broadcasted_iota(jnp.int32, (gw, _P), 1)
    base = b * _P

    def group(g, _):
        cxg = cx_ref[0, pl.ds(g * gw, gw), :]
        cyg = cy_ref[0, pl.ds(g * gw, gw), :]
        czg = cz_ref[0, pl.ds(g * gw, gw), :]
        dx = cxg - px
        dy = cyg - py
        dz = czg - pz
        d = dx * dx + dy * dy + dz * dz
        cols = []
        for _k in range(_K):
            mv = jnp.min(d, axis=1, keepdims=True)
            j = jnp.min(jnp.where(d == mv, lane, _P), axis=1, keepdims=True)
            cols.append(j)
            d = jnp.where(lane == j, jnp.inf, d)
        out = jnp.concatenate(cols, axis=1) + base
        col_ref[0, pl.ds(g * gw, gw), :] = out
        return 0

    lax.fori_loop(0, _M // gw, group, 0)


def _knn(px, py, pz, cx3, cy3, cz3, x, W, b, gamma, beta):
    c_spec = pl.BlockSpec((1, _M, 1), lambda b: (b, 0, 0))
    p_spec = pl.BlockSpec((1, 1, _P), lambda b: (b, 0, 0))
    v_spec = pl.BlockSpec((1, _COUT), lambda b: (0, 0))
    return pl.pallas_call(
        _knn_body,
        grid=(_B,),
        in_specs=[p_spec, p_spec, p_spec, c_spec, c_spec, c_spec,
                  pl.BlockSpec((_P, _CIN), lambda b: (b, 0)),
                  pl.BlockSpec((_CIN, _COUT), lambda b: (0, 0)),
                  v_spec, v_spec, v_spec],
        out_specs=(
            pl.BlockSpec((1, _M, _K), lambda b: (b, 0, 0)),
            pl.BlockSpec((_P, _COUT), lambda b: (b, 0)),
            pl.BlockSpec((8, _COUT), lambda b: (0, 0)),
        ),
        out_shape=(
            jax.ShapeDtypeStruct((_B, _M, _K), jnp.int32),
            jax.ShapeDtypeStruct((_N, _COUT), jnp.float32),
            jax.ShapeDtypeStruct((8, _COUT), jnp.float32),
        ),
    )(px.reshape(_B, 1, _P), py.reshape(_B, 1, _P), pz.reshape(_B, 1, _P),
      cx3, cy3, cz3, x, W, b.reshape(1, _COUT),
      gamma.reshape(1, _COUT), beta.reshape(1, _COUT))


def _pool(h, col, scale, shift):
    mesh = plsc.VectorSubcoreMesh(core_axis_name="c", subcore_axis_name="s")

    @functools.partial(
        pl.kernel,
        out_type=[
            jax.ShapeDtypeStruct((_B * _M, _COUT), jnp.float32),
            jax.ShapeDtypeStruct((_B * _M,), jnp.int32),
        ],
        mesh=mesh,
        scratch_types=[
            pltpu.VMEM((_NB, _K), jnp.int32),
            pltpu.VMEM((2, _K, _COUT), jnp.float32),
            pltpu.VMEM((_NB, _COUT), jnp.float32),
            pltpu.VMEM((_COUT,), jnp.float32),
            pltpu.VMEM((_COUT,), jnp.float32),
            pltpu.VMEM((_NB,), jnp.int32),
            pltpu.SemaphoreType.DMA,
            pltpu.SemaphoreType.DMA,
        ],
    )
    def k(h_hbm, col_hbm, sc_hbm, sh_hbm, out_hbm, sb_hbm,
          idx_v, rows_v, out_v, sc_v, sh_v, sb_v, sem0, sem1):
        wid = lax.axis_index("s") * 2 + lax.axis_index("c")
        base = wid * _NB
        pltpu.sync_copy(col_hbm.at[pl.ds(base, _NB)], idx_v)
        pltpu.sync_copy(sc_hbm, sc_v)
        pltpu.sync_copy(sh_hbm, sh_v)

        def _compute(i, buf):
            for c in range(_COUT // 16):
                sl = pl.ds(c * 16, 16)
                acc = buf[0, sl]
                for r in range(1, _K):
                    acc = jnp.maximum(acc, buf[r, sl])
                y = jnp.maximum(acc * sc_v[sl] + sh_v[sl], 0.0)
                out_v[i, sl] = y

        pltpu.async_copy(h_hbm.at[idx_v.at[0]], rows_v.at[0], sem0)

        @pl.loop(0, _NB, step=2)
        def _(i):
            pltpu.async_copy(h_hbm.at[idx_v.at[i + 1]], rows_v.at[1], sem1)
            pltpu.make_async_copy(
                h_hbm.at[idx_v.at[i]], rows_v.at[0], sem0).wait()
            _compute(i, rows_v.at[0])

            @pl.when(i + 2 < _NB)
            def _():
                pltpu.async_copy(
                    h_hbm.at[idx_v.at[i + 2]], rows_v.at[0], sem0)

            pltpu.make_async_copy(
                h_hbm.at[idx_v.at[i + 1]], rows_v.at[1], sem1).wait()
            _compute(i + 1, rows_v.at[1])

        @pl.loop(0, _NB, step=16)
        def _(i):
            v = lax.broadcasted_iota(jnp.int32, (16,), 0)
            sb_v[pl.ds(i, 16)] = lax.shift_right_logical(v + (base + i), 9)

        pltpu.sync_copy(out_v, out_hbm.at[pl.ds(base, _NB)])
        pltpu.sync_copy(sb_v, sb_hbm.at[pl.ds(base, _NB)])

    return k(h, col, scale, shift)


def kernel(x, pos, batch, W, b, gamma, beta):
    pos3 = pos.reshape(_B, _P, 3)
    px = pos3[:, :, 0]
    py = pos3[:, :, 1]
    pz = pos3[:, :, 2]
    cx, cy, cz = _fps(px, py, pz)
    col3, h, ss = _knn(px, py, pz,
                       cx.reshape(_B, _M, 1), cy.reshape(_B, _M, 1),
                       cz.reshape(_B, _M, 1), x, W, b, gamma, beta)
    col = col3.reshape(_B * _M, _K)
    x_out, sub_batch = _pool(h, col, ss[0], ss[1])
    sub_pos = jnp.stack([cx, cy, cz], axis=-1).reshape(_B * _M, 3)
    return (x_out, sub_pos, sub_batch)


# speedup vs baseline: 1.2800x; 1.0910x over previous
import functools

import jax
import jax.numpy as jnp
from jax import lax
from jax.experimental import pallas as pl
from jax.experimental.pallas import tpu as pltpu
from jax.experimental.pallas import tpu_sc as plsc

_B, _P, _CIN, _COUT = 8, 2048, 128, 256
_M, _K = 512, 16
_N = _B * _P
_EPS = 1e-5
_NW = 32
_NB = (_B * _M) // _NW


_NC = _P // 128


def _fps_body(px_ref, py_ref, pz_ref, cx_ref, cy_ref, cz_ref):
    ixs = [px_ref[:, 128 * c:128 * (c + 1)] for c in range(_NC)]
    iys = [py_ref[:, 128 * c:128 * (c + 1)] for c in range(_NC)]
    izs = [pz_ref[:, 128 * c:128 * (c + 1)] for c in range(_NC)]
    lanef = lax.broadcasted_iota(jnp.int32, (_B, 128), 1).astype(jnp.float32)
    mlane = lax.broadcasted_iota(jnp.int32, (_B, _M), 1)
    x0 = ixs[0][:, 0:1]
    y0 = iys[0][:, 0:1]
    z0 = izs[0][:, 0:1]
    dists = []
    for c in range(_NC):
        d0x = ixs[c] - x0
        d0y = iys[c] - y0
        d0z = izs[c] - z0
        dists.append(d0x * d0x + d0y * d0y + d0z * d0z)
    cx = jnp.where(mlane == 0, jnp.broadcast_to(x0, (_B, _M)), 0.0)
    cy = jnp.where(mlane == 0, jnp.broadcast_to(y0, (_B, _M)), 0.0)
    cz = jnp.where(mlane == 0, jnp.broadcast_to(z0, (_B, _M)), 0.0)

    def _tmax(vals):
        while len(vals) > 1:
            vals = [jnp.maximum(vals[2 * i], vals[2 * i + 1])
                    for i in range(len(vals) // 2)]
        return vals[0]

    def _pick(mask_list, vals, fill):
        sel = [jnp.where(mask_list[c], vals[c], fill) for c in range(_NC)]
        return _tmax(sel)

    def body(i, carry):
        dists, cx, cy, cz = carry
        mc = [jnp.max(dists[c], axis=1, keepdims=True) for c in range(_NC)]
        m = _tmax(list(mc))
        cidx = [jnp.where(mc[c] == m, c, _NC) for c in range(_NC)]
        cstar = cidx[0]
        for c in range(1, _NC):
            cstar = jnp.minimum(cstar, cidx[c])
        cmask = [cstar == c for c in range(_NC)]
        vstar = _pick(cmask, dists, -1.0)
        lstar = jnp.min(jnp.where(vstar == m, lanef, 128.0), axis=1,
                        keepdims=True)
        selm = lanef == lstar
        pxs = _pick(cmask, ixs, -1.0)
        pys = _pick(cmask, iys, -1.0)
        pzs = _pick(cmask, izs, -1.0)
        lx = jnp.max(jnp.where(selm, pxs, -1.0), axis=1, keepdims=True)
        ly = jnp.max(jnp.where(selm, pys, -1.0), axis=1, keepdims=True)
        lz = jnp.max(jnp.where(selm, pzs, -1.0), axis=1, keepdims=True)
        new = []
        for c in range(_NC):
            dx = ixs[c] - lx
            dy = iys[c] - ly
            dz = izs[c] - lz
            dn = dx * dx + dy * dy + dz * dz
            new.append(jnp.minimum(dists[c], dn))
        upd = mlane == i
        cx = jnp.where(upd, lx, cx)
        cy = jnp.where(upd, ly, cy)
        cz = jnp.where(upd, lz, cz)
        return (new, cx, cy, cz)

    _, cx, cy, cz = lax.fori_loop(1, _M, body, (dists, cx, cy, cz))
    cx_ref[...] = cx
    cy_ref[...] = cy
    cz_ref[...] = cz


def _fps(px, py, pz):
    out = jax.ShapeDtypeStruct((_B, _M), jnp.float32)
    return pl.pallas_call(
        _fps_body,
        out_shape=(out, out, out),
    )(px, py, pz)


def _knn_body(px_ref, py_ref, pz_ref, cx_ref, cy_ref, cz_ref,
              x_ref, w_ref, b_ref, g_ref, be_ref, col_ref, h_ref, ss_ref):
    b = pl.program_id(0)
    h = jnp.dot(x_ref[...], w_ref[...],
                preferred_element_type=jnp.float32,
                precision=lax.Precision.HIGHEST) + b_ref[...]
    h_ref[...] = h
    s1 = jnp.sum(h, axis=0, keepdims=True)
    s2 = jnp.sum(h * h, axis=0, keepdims=True)

    @pl.when(b == 0)
    def _():
        ss_ref[2:3, :] = s1
        ss_ref[3:4, :] = s2

    @pl.when(b > 0)
    def _():
        ss_ref[2:3, :] = ss_ref[2:3, :] + s1
        ss_ref[3:4, :] = ss_ref[3:4, :] + s2

    @pl.when(b == _B - 1)
    def _():
        mu = ss_ref[2:3, :] / _N
        var = ss_ref[3:4, :] / _N - mu * mu
        scale = g_ref[...] / jnp.sqrt(var + _EPS)
        ss_ref[0:1, :] = scale
        ss_ref[1:2, :] = be_ref[...] - mu * scale

    px = px_ref[0]
    py = py_ref[0]
    pz = pz_ref[0]
    gw = 512
    lane = lax.broadcasted_iota(jnp.int32, (gw, _P), 1)
    base = b * _P

    def group(g, _):
        cxg = cx_ref[0, pl.ds(g * gw, gw), :]
        cyg = cy_ref[0, pl.ds(g * gw, gw), :]
        czg = cz_ref[0, pl.ds(g * gw, gw), :]
        dx = cxg - px
        dy = cyg - py
        dz = czg - pz
        d = dx * dx + dy * dy + dz * dz
        cols = []
        for _k in range(_K):
            mv = jnp.min(d, axis=1, keepdims=True)
            j = jnp.min(jnp.where(d == mv, lane, _P), axis=1, keepdims=True)
            cols.append(j)
            d = jnp.where(lane == j, jnp.inf, d)
        out = jnp.concatenate(cols, axis=1) + base
        col_ref[0, pl.ds(g * gw, gw), :] = out
        return 0

    lax.fori_loop(0, _M // gw, group, 0)


def _knn(px, py, pz, cx3, cy3, cz3, x, W, b, gamma, beta):
    c_spec = pl.BlockSpec((1, _M, 1), lambda b: (b, 0, 0))
    p_spec = pl.BlockSpec((1, 1, _P), lambda b: (b, 0, 0))
    v_spec = pl.BlockSpec((1, _COUT), lambda b: (0, 0))
    return pl.pallas_call(
        _knn_body,
        grid=(_B,),
        in_specs=[p_spec, p_spec, p_spec, c_spec, c_spec, c_spec,
                  pl.BlockSpec((_P, _CIN), lambda b: (b, 0)),
                  pl.BlockSpec((_CIN, _COUT), lambda b: (0, 0)),
                  v_spec, v_spec, v_spec],
        out_specs=(
            pl.BlockSpec((1, _M, _K), lambda b: (b, 0, 0)),
            pl.BlockSpec((_P, _COUT), lambda b: (b, 0)),
            pl.BlockSpec((8, _COUT), lambda b: (0, 0)),
        ),
        out_shape=(
            jax.ShapeDtypeStruct((_B, _M, _K), jnp.int32),
            jax.ShapeDtypeStruct((_N, _COUT), jnp.float32),
            jax.ShapeDtypeStruct((8, _COUT), jnp.float32),
        ),
    )(px.reshape(_B, 1, _P), py.reshape(_B, 1, _P), pz.reshape(_B, 1, _P),
      cx3, cy3, cz3, x, W, b.reshape(1, _COUT),
      gamma.reshape(1, _COUT), beta.reshape(1, _COUT))


def _pool(h, col, scale, shift):
    mesh = plsc.VectorSubcoreMesh(core_axis_name="c", subcore_axis_name="s")

    @functools.partial(
        pl.kernel,
        out_type=[
            jax.ShapeDtypeStruct((_B * _M, _COUT), jnp.float32),
            jax.ShapeDtypeStruct((_B * _M,), jnp.int32),
        ],
        mesh=mesh,
        scratch_types=[
            pltpu.VMEM((_NB, _K), jnp.int32),
            pltpu.VMEM((2, _K, _COUT), jnp.float32),
            pltpu.VMEM((_NB, _COUT), jnp.float32),
            pltpu.VMEM((_COUT,), jnp.float32),
            pltpu.VMEM((_COUT,), jnp.float32),
            pltpu.VMEM((_NB,), jnp.int32),
            pltpu.SemaphoreType.DMA,
            pltpu.SemaphoreType.DMA,
        ],
    )
    def k(h_hbm, col_hbm, sc_hbm, sh_hbm, out_hbm, sb_hbm,
          idx_v, rows_v, out_v, sc_v, sh_v, sb_v, sem0, sem1):
        wid = lax.axis_index("s") * 2 + lax.axis_index("c")
        base = wid * _NB
        pltpu.sync_copy(col_hbm.at[pl.ds(base, _NB)], idx_v)
        pltpu.sync_copy(sc_hbm, sc_v)
        pltpu.sync_copy(sh_hbm, sh_v)

        def _compute(i, buf):
            for c in range(_COUT // 16):
                sl = pl.ds(c * 16, 16)
                acc = buf[0, sl]
                for r in range(1, _K):
                    acc = jnp.maximum(acc, buf[r, sl])
                y = jnp.maximum(acc * sc_v[sl] + sh_v[sl], 0.0)
                out_v[i, sl] = y

        pltpu.async_copy(h_hbm.at[idx_v.at[0]], rows_v.at[0], sem0)

        @pl.loop(0, _NB, step=2)
        def _(i):
            pltpu.async_copy(h_hbm.at[idx_v.at[i + 1]], rows_v.at[1], sem1)
            pltpu.make_async_copy(
                h_hbm.at[idx_v.at[i]], rows_v.at[0], sem0).wait()
            _compute(i, rows_v.at[0])

            @pl.when(i + 2 < _NB)
            def _():
                pltpu.async_copy(
                    h_hbm.at[idx_v.at[i + 2]], rows_v.at[0], sem0)

            pltpu.make_async_copy(
                h_hbm.at[idx_v.at[i + 1]], rows_v.at[1], sem1).wait()
            _compute(i + 1, rows_v.at[1])

        @pl.loop(0, _NB, step=16)
        def _(i):
            v = lax.broadcasted_iota(jnp.int32, (16,), 0)
            sb_v[pl.ds(i, 16)] = lax.shift_right_logical(v + (base + i), 9)

        pltpu.sync_copy(out_v, out_hbm.at[pl.ds(base, _NB)])
        pltpu.sync_copy(sb_v, sb_hbm.at[pl.ds(base, _NB)])

    return k(h, col, scale, shift)


def kernel(x, pos, batch, W, b, gamma, beta):
    pos3 = pos.reshape(_B, _P, 3)
    px = pos3[:, :, 0]
    py = pos3[:, :, 1]
    pz = pos3[:, :, 2]
    cx, cy, cz = _fps(px, py, pz)
    col3, h, ss = _knn(px, py, pz,
                       cx.reshape(_B, _M, 1), cy.reshape(_B, _M, 1),
                       cz.reshape(_B, _M, 1), x, W, b, gamma, beta)
    col = col3.reshape(_B * _M, _K)
    x_out, sub_batch = _pool(h, col, ss[0], ss[1])
    sub_pos = jnp.stack([cx, cy, cz], axis=-1).reshape(_B * _M, 3)
    return (x_out, sub_pos, sub_batch)
